# baseline (speedup 1.0000x reference)
import functools

import jax
import jax.numpy as jnp
from jax import lax
from jax.experimental import pallas as pl
from jax.experimental.pallas import tpu as pltpu
from jax.experimental.pallas import tpu_sc as plsc

N = 10000
E = 320000
D = 128

NC = 2
NS = 16
NW = NC * NS
L = 16

K = 128
CH = 160
W = 8
NCHUNK = NS * CH
E_PAD = NCHUNK * K
N_PAD = 10240
RPW = N_PAD // NS

ROW_T = 400
ROW_U = 512


def _prep_body(x_ref, wp_ref, bp_ref, out_ref):
    t = jnp.dot(x_ref[...], wp_ref[...], preferred_element_type=jnp.float32)
    out_ref[...] = jnp.maximum(t + bp_ref[...], 0.0)


def _prepare_table(x, Wp, bp):
    return pl.pallas_call(
        _prep_body,
        grid=(N // ROW_T,),
        in_specs=[
            pl.BlockSpec((ROW_T, D), lambda i: (i, 0)),
            pl.BlockSpec((D, D), lambda i: (0, 0)),
            pl.BlockSpec((1, D), lambda i: (0, 0)),
        ],
        out_specs=pl.BlockSpec((ROW_T, D), lambda i: (i, 0)),
        out_shape=jax.ShapeDtypeStruct((N, D), jnp.float32),
    )(x, Wp, bp.reshape(1, D))


def _sc_body(table_hbm, src_hbm, dst_hbm, part_hbm, cnt_hbm,
             src_v, dst_v, buf0, buf1, hist, acc_sh,
             sem0, sem1):
    c = lax.axis_index("c")
    s = lax.axis_index("s")
    base = s * RPW
    cb = s * CH
    ones16 = jnp.ones((L,), jnp.float32)

    def count(q):
        for t in range(K // L):
            ids = dst_v[q, pl.ds(t * L, L)]
            plsc.addupdate_scatter(
                hist, [lax.shift_right_logical(ids, 7), ids & (K - 1)],
                ones16)

    def gather(q, buf, sem):
        pltpu.async_copy(table_hbm.at[src_v.at[q]], buf, sem)

    def consume(q, buf, sem):
        pltpu.make_async_copy(table_hbm.at[src_v.at[q]], buf, sem).wait()
        pltpu.sync_copy(buf, acc_sh.at[dst_v.at[q]], add=True)

    @pl.when(c == 0)
    def _payload():
        def zrow(i, carry):
            def zcol(g, carry2):
                buf0[i, pl.ds(g * L, L)] = jnp.zeros((L,), jnp.float32)
                return carry2
            return lax.fori_loop(0, D // L, zcol, carry)
        lax.fori_loop(0, K, zrow, 0)
        for b in range(RPW // K):
            pltpu.sync_copy(buf0, acc_sh.at[pl.ds(base + b * K, K)])
        plsc.subcore_barrier()

        def win_body(win, carry):
            pltpu.sync_copy(src_hbm.at[cb // W + win], src_v)
            pltpu.sync_copy(dst_hbm.at[cb // W + win], dst_v)
            gather(0, buf0, sem0)
            for q in range(W):
                if q + 1 < W:
                    gather(q + 1, (buf1, buf0)[q % 2], (sem1, sem0)[q % 2])
                consume(q, (buf0, buf1)[q % 2], (sem0, sem1)[q % 2])
            return carry
        lax.fori_loop(0, CH // W, win_body, 0)
        plsc.subcore_barrier()

        for b in range(RPW // K):
            r0 = base + b * K
            pltpu.sync_copy(acc_sh.at[pl.ds(r0, K)],
                            part_hbm.at[pl.ds(r0, K)])

    @pl.when(c == 1)
    def _counts():
        def zhist(i, carry):
            def zcol(g, carry2):
                hist[i, pl.ds(g * L, L)] = jnp.zeros((L,), jnp.float32)
                return carry2
            return lax.fori_loop(0, K // L, zcol, carry)
        lax.fori_loop(0, N_PAD // K, zhist, 0)

        def cwin_body(win, carry):
            pltpu.sync_copy(dst_hbm.at[cb // W + win], dst_v)
            for q in range(W):
                count(q)
            return carry
        lax.fori_loop(0, CH // W, cwin_body, 0)
        pltpu.sync_copy(hist, cnt_hbm.at[s])


_sc_aggregate = functools.partial(
    pl.kernel,
    mesh=plsc.VectorSubcoreMesh(core_axis_name="c", subcore_axis_name="s"),
    out_type=(
        jax.ShapeDtypeStruct((N_PAD, D), jnp.float32),
        jax.ShapeDtypeStruct((NS, N_PAD // K, K), jnp.float32),
    ),
    scratch_types=[
        pltpu.VMEM((W, K), jnp.int32),
        pltpu.VMEM((W, K), jnp.int32),
        pltpu.VMEM((K, D), jnp.float32),
        pltpu.VMEM((K, D), jnp.float32),
        pltpu.VMEM((N_PAD // K, K), jnp.float32),
        pltpu.VMEM_SHARED((N_PAD, D), jnp.float32),
        pltpu.SemaphoreType.DMA,
        pltpu.SemaphoreType.DMA,
    ],
    compiler_params=pltpu.CompilerParams(needs_layout_passes=False),
)(_sc_body)


def _upd_body(x_ref, part_ref, cnt_ref, wu_ref, bu_ref, out_ref):
    ssum = part_ref[...]
    cnt = jnp.sum(cnt_ref[...], axis=0)
    recip = 1.0 / jnp.maximum(cnt, 1.0)
    rec2d = jnp.transpose(jnp.broadcast_to(recip, (D, ROW_U)), (1, 0))
    agg = ssum * rec2d
    h = jnp.dot(x_ref[...], wu_ref[:D], preferred_element_type=jnp.float32)
    h = h + jnp.dot(agg, wu_ref[D:], preferred_element_type=jnp.float32)
    out_ref[...] = jnp.maximum(h + bu_ref[...], 0.0)


def _update(x, part, cnt, Wu, bu):
    grid = -(-N // ROW_U)
    return pl.pallas_call(
        _upd_body,
        grid=(grid,),
        in_specs=[
            pl.BlockSpec((ROW_U, D), lambda i: (i, 0)),
            pl.BlockSpec((ROW_U, D), lambda i: (i, 0)),
            pl.BlockSpec((NS, ROW_U), lambda i: (0, i)),
            pl.BlockSpec((2 * D, D), lambda i: (0, 0)),
            pl.BlockSpec((1, D), lambda i: (0, 0)),
        ],
        out_specs=pl.BlockSpec((ROW_U, D), lambda i: (i, 0)),
        out_shape=jax.ShapeDtypeStruct((N, D), jnp.float32),
    )(x, part, cnt.reshape(NS, N_PAD), Wu, bu.reshape(1, D))


def kernel(node_repesentations, edges, Wp, bp, Wu, bu):
    x = node_repesentations.reshape(N, D)
    src = edges[1]
    dst = edges[0]
    pad = E_PAD - E
    src_p = jnp.concatenate(
        [src, jnp.zeros((pad,), jnp.int32)]).reshape(NCHUNK // W, W, K)
    dst_p = jnp.concatenate(
        [dst, jnp.full((pad,), N, jnp.int32)]).reshape(NCHUNK // W, W, K)
    table = _prepare_table(x, Wp, bp)
    part, cnt = _sc_aggregate(table, src_p, dst_p)
    out = _update(x, part, cnt, Wu, bu)
    return out.reshape(1, 1, N, D)


# speedup vs baseline: 1.1980x; 1.1980x over previous
import functools

import jax
import jax.numpy as jnp
from jax import lax
from jax.experimental import pallas as pl
from jax.experimental.pallas import tpu as pltpu
from jax.experimental.pallas import tpu_sc as plsc

N = 10000
E = 320000
D = 128

NC = 2
NS = 16
NW = NC * NS
L = 16

K = 64
NBUF = 4
CH0 = 256
CH1 = 64
W = 16
NCHUNK = NS * (CH0 + CH1)
E_PAD = NCHUNK * K
N_PAD = 10240
RPW = N_PAD // NS

ROW_T = 400
ROW_U = 512


def _prep_body(x_ref, wp_ref, bp_ref, out_ref):
    t = jnp.dot(x_ref[...], wp_ref[...], preferred_element_type=jnp.float32)
    out_ref[...] = jnp.maximum(t + bp_ref[...], 0.0)


def _prepare_table(x, Wp, bp):
    return pl.pallas_call(
        _prep_body,
        grid=(N // ROW_T,),
        in_specs=[
            pl.BlockSpec((ROW_T, D), lambda i: (i, 0)),
            pl.BlockSpec((D, D), lambda i: (0, 0)),
            pl.BlockSpec((1, D), lambda i: (0, 0)),
        ],
        out_specs=pl.BlockSpec((ROW_T, D), lambda i: (i, 0)),
        out_shape=jax.ShapeDtypeStruct((N, D), jnp.float32),
    )(x, Wp, bp.reshape(1, D))


def _sc_body(table_hbm, src_hbm, dst_hbm, part_hbm, cnt_hbm,
             src_v, dst_v, buf0, buf1, buf2, buf3, hist, acc_sh,
             sem0, sem1, sem2, sem3):
    c = lax.axis_index("c")
    s = lax.axis_index("s")
    cb = jnp.where(c == 0, s * CH0, NS * CH0 + s * CH1)
    nwin = jnp.where(c == 0, CH0 // W, CH1 // W)

    def zrow(i, carry):
        def zcol(g, carry2):
            buf0[i, pl.ds(g * L, L)] = jnp.zeros((L,), jnp.float32)
            return carry2
        return lax.fori_loop(0, D // L, zcol, carry)
    lax.fori_loop(0, K, zrow, 0)
    bufs = (buf0, buf1, buf2, buf3)
    sems = (sem0, sem1, sem2, sem3)

    def zhist(i, carry):
        def zcol(g, carry2):
            hist[i, pl.ds(g * L, L)] = jnp.zeros((L,), jnp.float32)
            return carry2
        return lax.fori_loop(0, 128 // L, zcol, carry)
    lax.fori_loop(0, N_PAD // 128, zhist, 0)

    base = s * RPW
    for b in range(RPW // K):
        pltpu.sync_copy(buf0, acc_sh.at[pl.ds(base + b * K, K)])
    plsc.subcore_barrier()

    ones16 = jnp.ones((L,), jnp.float32)

    def count(q):
        for t in range(K // L):
            ids = dst_v[q, pl.ds(t * L, L)]
            plsc.addupdate_scatter(
                hist, [lax.shift_right_logical(ids, 7), ids & 127],
                ones16)

    def gather(q, buf, sem):
        pltpu.async_copy(table_hbm.at[src_v.at[q]], buf, sem)

    def consume(q, buf, sem):
        pltpu.make_async_copy(table_hbm.at[src_v.at[q]], buf, sem).wait()
        pltpu.sync_copy(buf, acc_sh.at[dst_v.at[q]], add=True)

    def win_body(win, carry):
        pltpu.sync_copy(src_hbm.at[cb // W + win], src_v)
        pltpu.sync_copy(dst_hbm.at[cb // W + win], dst_v)
        for d in range(NBUF - 1):
            gather(d, bufs[d], sems[d])
        for q in range(W):
            nq = q + NBUF - 1
            if nq < W:
                gather(nq, bufs[nq % NBUF], sems[nq % NBUF])
            count(q)
            consume(q, bufs[q % NBUF], sems[q % NBUF])
        return carry
    lax.fori_loop(0, nwin, win_body, 0)

    pltpu.sync_copy(hist, cnt_hbm.at[c, s])
    plsc.subcore_barrier()
    for b in range(RPW // K):
        r0 = base + b * K
        pltpu.sync_copy(acc_sh.at[pl.ds(r0, K)], part_hbm.at[c, pl.ds(r0, K)])


_sc_aggregate = functools.partial(
    pl.kernel,
    mesh=plsc.VectorSubcoreMesh(core_axis_name="c", subcore_axis_name="s"),
    out_type=(
        jax.ShapeDtypeStruct((NC, N_PAD, D), jnp.float32),
        jax.ShapeDtypeStruct((NC, NS, N_PAD // 128, 128), jnp.float32),
    ),
    scratch_types=[
        pltpu.VMEM((W, K), jnp.int32),
        pltpu.VMEM((W, K), jnp.int32),
        pltpu.VMEM((K, D), jnp.float32),
        pltpu.VMEM((K, D), jnp.float32),
        pltpu.VMEM((K, D), jnp.float32),
        pltpu.VMEM((K, D), jnp.float32),
        pltpu.VMEM((N_PAD // 128, 128), jnp.float32),
        pltpu.VMEM_SHARED((N_PAD, D), jnp.float32),
        pltpu.SemaphoreType.DMA,
        pltpu.SemaphoreType.DMA,
        pltpu.SemaphoreType.DMA,
        pltpu.SemaphoreType.DMA,
    ],
    compiler_params=pltpu.CompilerParams(needs_layout_passes=False),
)(_sc_body)


def _upd_body(x_ref, part_ref, cnt_ref, wu_ref, bu_ref, out_ref):
    ssum = part_ref[0] + part_ref[1]
    cnt = jnp.sum(cnt_ref[...], axis=0)
    recip = 1.0 / jnp.maximum(cnt, 1.0)
    rec2d = jnp.transpose(jnp.broadcast_to(recip, (D, ROW_U)), (1, 0))
    agg = ssum * rec2d
    h = jnp.dot(x_ref[...], wu_ref[:D], preferred_element_type=jnp.float32)
    h = h + jnp.dot(agg, wu_ref[D:], preferred_element_type=jnp.float32)
    out_ref[...] = jnp.maximum(h + bu_ref[...], 0.0)


def _update(x, part, cnt, Wu, bu):
    grid = -(-N // ROW_U)
    return pl.pallas_call(
        _upd_body,
        grid=(grid,),
        in_specs=[
            pl.BlockSpec((ROW_U, D), lambda i: (i, 0)),
            pl.BlockSpec((NC, ROW_U, D), lambda i: (0, i, 0)),
            pl.BlockSpec((NW, ROW_U), lambda i: (0, i)),
            pl.BlockSpec((2 * D, D), lambda i: (0, 0)),
            pl.BlockSpec((1, D), lambda i: (0, 0)),
        ],
        out_specs=pl.BlockSpec((ROW_U, D), lambda i: (i, 0)),
        out_shape=jax.ShapeDtypeStruct((N, D), jnp.float32),
    )(x, part, cnt.reshape(NW, N_PAD), Wu, bu.reshape(1, D))


def kernel(node_repesentations, edges, Wp, bp, Wu, bu):
    x = node_repesentations.reshape(N, D)
    src = edges[1]
    dst = edges[0]
    pad = E_PAD - E
    src_p = jnp.concatenate(
        [src, jnp.zeros((pad,), jnp.int32)]).reshape(NCHUNK // W, W, K)
    dst_p = jnp.concatenate(
        [dst, jnp.full((pad,), N, jnp.int32)]).reshape(NCHUNK // W, W, K)
    table = _prepare_table(x, Wp, bp)
    part, cnt = _sc_aggregate(table, src_p, dst_p)
    out = _update(x, part, cnt, Wu, bu)
    return out.reshape(1, 1, N, D)


# speedup vs baseline: 1.3808x; 1.1525x over previous
import functools

import jax
import jax.numpy as jnp
from jax import lax
from jax.experimental import pallas as pl
from jax.experimental.pallas import tpu as pltpu
from jax.experimental.pallas import tpu_sc as plsc

N = 10000
E = 320000
D = 128

NC = 2
NS = 16
NW = NC * NS
L = 16

K = 128
CH0 = 144
CH1 = 16
W = 8
NCHUNK = NS * (CH0 + CH1)
E_PAD = NCHUNK * K
N_PAD = 10240
RPW = N_PAD // NS

ROW_T = 400
ROW_U = 512


def _prep_body(x_ref, wp_ref, bp_ref, out_ref):
    t = jnp.dot(x_ref[...], wp_ref[...], preferred_element_type=jnp.float32)
    out_ref[...] = jnp.maximum(t + bp_ref[...], 0.0)


def _prepare_table(x, Wp, bp):
    return pl.pallas_call(
        _prep_body,
        grid=(N // ROW_T,),
        in_specs=[
            pl.BlockSpec((ROW_T, D), lambda i: (i, 0)),
            pl.BlockSpec((D, D), lambda i: (0, 0)),
            pl.BlockSpec((1, D), lambda i: (0, 0)),
        ],
        out_specs=pl.BlockSpec((ROW_T, D), lambda i: (i, 0)),
        out_shape=jax.ShapeDtypeStruct((N, D), jnp.float32),
    )(x, Wp, bp.reshape(1, D))


def _sc_body(table_hbm, src_hbm, dst_hbm, part_hbm, cnt_hbm,
             src_v, dst_v, buf0, buf1, hist, acc_sh,
             sem0, sem1):
    c = lax.axis_index("c")
    s = lax.axis_index("s")
    cb = jnp.where(c == 0, s * CH0, NS * CH0 + s * CH1)
    nwin = jnp.where(c == 0, CH0 // W, CH1 // W)

    def zrow(i, carry):
        def zcol(g, carry2):
            buf0[i, pl.ds(g * L, L)] = jnp.zeros((L,), jnp.float32)
            return carry2
        return lax.fori_loop(0, D // L, zcol, carry)
    lax.fori_loop(0, K, zrow, 0)

    def zhist(i, carry):
        def zcol(g, carry2):
            hist[i, pl.ds(g * L, L)] = jnp.zeros((L,), jnp.float32)
            return carry2
        return lax.fori_loop(0, K // L, zcol, carry)
    lax.fori_loop(0, N_PAD // K, zhist, 0)

    base = s * RPW
    for b in range(RPW // K):
        pltpu.sync_copy(buf0, acc_sh.at[pl.ds(base + b * K, K)])
    plsc.subcore_barrier()

    ones16 = jnp.ones((L,), jnp.float32)

    def count(q):
        for t in range(K // L):
            ids = dst_v[q, pl.ds(t * L, L)]
            plsc.addupdate_scatter(
                hist, [lax.shift_right_logical(ids, 7), ids & (K - 1)],
                ones16)

    def gather(q, buf, sem):
        pltpu.async_copy(table_hbm.at[src_v.at[q]], buf, sem)

    def consume(q, buf, sem):
        pltpu.make_async_copy(table_hbm.at[src_v.at[q]], buf, sem).wait()
        pltpu.sync_copy(buf, acc_sh.at[dst_v.at[q]], add=True)

    def win_body(win, carry):
        pltpu.sync_copy(src_hbm.at[cb // W + win], src_v)
        pltpu.sync_copy(dst_hbm.at[cb // W + win], dst_v)
        gather(0, buf0, sem0)
        for q in range(W):
            if q + 1 < W:
                gather(q + 1, (buf1, buf0)[q % 2], (sem1, sem0)[q % 2])
            count(q)
            consume(q, (buf0, buf1)[q % 2], (sem0, sem1)[q % 2])
        return carry
    lax.fori_loop(0, nwin, win_body, 0)

    pltpu.sync_copy(hist, cnt_hbm.at[c, s])
    plsc.subcore_barrier()
    for b in range(RPW // K):
        r0 = base + b * K
        pltpu.sync_copy(acc_sh.at[pl.ds(r0, K)], part_hbm.at[c, pl.ds(r0, K)])


_sc_aggregate = functools.partial(
    pl.kernel,
    mesh=plsc.VectorSubcoreMesh(core_axis_name="c", subcore_axis_name="s"),
    out_type=(
        jax.ShapeDtypeStruct((NC, N_PAD, D), jnp.float32),
        jax.ShapeDtypeStruct((NC, NS, N_PAD // K, K), jnp.float32),
    ),
    scratch_types=[
        pltpu.VMEM((W, K), jnp.int32),
        pltpu.VMEM((W, K), jnp.int32),
        pltpu.VMEM((K, D), jnp.float32),
        pltpu.VMEM((K, D), jnp.float32),
        pltpu.VMEM((N_PAD // K, K), jnp.float32),
        pltpu.VMEM_SHARED((N_PAD, D), jnp.float32),
        pltpu.SemaphoreType.DMA,
        pltpu.SemaphoreType.DMA,
    ],
    compiler_params=pltpu.CompilerParams(needs_layout_passes=False),
)(_sc_body)


def _upd_body(x_ref, part_ref, cnt_ref, wu_ref, bu_ref, out_ref):
    ssum = part_ref[0] + part_ref[1]
    cnt = jnp.sum(cnt_ref[...], axis=0)
    recip = 1.0 / jnp.maximum(cnt, 1.0)
    rec2d = jnp.transpose(jnp.broadcast_to(recip, (D, ROW_U)), (1, 0))
    agg = ssum * rec2d
    h = jnp.dot(x_ref[...], wu_ref[:D], preferred_element_type=jnp.float32)
    h = h + jnp.dot(agg, wu_ref[D:], preferred_element_type=jnp.float32)
    out_ref[...] = jnp.maximum(h + bu_ref[...], 0.0)


def _update(x, part, cnt, Wu, bu):
    grid = -(-N // ROW_U)
    return pl.pallas_call(
        _upd_body,
        grid=(grid,),
        in_specs=[
            pl.BlockSpec((ROW_U, D), lambda i: (i, 0)),
            pl.BlockSpec((NC, ROW_U, D), lambda i: (0, i, 0)),
            pl.BlockSpec((NW, ROW_U), lambda i: (0, i)),
            pl.BlockSpec((2 * D, D), lambda i: (0, 0)),
            pl.BlockSpec((1, D), lambda i: (0, 0)),
        ],
        out_specs=pl.BlockSpec((ROW_U, D), lambda i: (i, 0)),
        out_shape=jax.ShapeDtypeStruct((N, D), jnp.float32),
    )(x, part, cnt.reshape(NW, N_PAD), Wu, bu.reshape(1, D))


def kernel(node_repesentations, edges, Wp, bp, Wu, bu):
    x = node_repesentations.reshape(N, D)
    src = edges[1]
    dst = edges[0]
    pad = E_PAD - E
    src_p = jnp.concatenate(
        [src, jnp.zeros((pad,), jnp.int32)]).reshape(NCHUNK // W, W, K)
    dst_p = jnp.concatenate(
        [dst, jnp.full((pad,), N, jnp.int32)]).reshape(NCHUNK // W, W, K)
    table = _prepare_table(x, Wp, bp)
    part, cnt = _sc_aggregate(table, src_p, dst_p)
    out = _update(x, part, cnt, Wu, bu)
    return out.reshape(1, 1, N, D)
